# baseline (speedup 1.0000x reference)
import functools

import jax
import jax.numpy as jnp
from jax import lax
from jax.experimental import pallas as pl
from jax.experimental.pallas import tpu as pltpu
from jax.experimental.pallas import tpu_sc as plsc

_N = 10000
_E = 320000
_D = 128
_H = 128
_DE = 16
_CP = 16

_NC = 2
_NS = 16
_NW = _NC * _NS
_TPW = _E // _NW
_CH = 80
_NCH = _TPW // _CH
_NP = 10240
_NPS = _NP // _NS
_SCH = 40
_SNCH = _TPW // _SCH
_CW = 8
_NQ = _NP // 16
_NQS = _NQ // _NS


def _sc_gather_body(p_hbm, q_hbm, cx_hbm, cy_hbm, cz_hbm,
                    idxrf_hbm, idxcf_hbm,
                    g_hbm, cd_hbm,
                    idxrf_v, idxcf_v,
                    cx_v, cy_v, cz_v, bufp, bufq, cdbuf,
                    sem0, sem1, sem2, sem3):
  c = lax.axis_index("c")
  s = lax.axis_index("s")
  wid = s * _NC + c
  base = wid * _TPW
  pltpu.sync_copy(idxrf_hbm.at[wid], idxrf_v)
  pltpu.sync_copy(idxcf_hbm.at[wid], idxcf_v)
  pltpu.sync_copy(cx_hbm, cx_v)
  pltpu.sync_copy(cy_hbm, cy_v)
  pltpu.sync_copy(cz_hbm, cz_v)

  b0p, b0q = bufp.at[0], bufq.at[0]
  b1p, b1q = bufp.at[1], bufq.at[1]

  def issue(cc, bp, bq, semp, semq):
    pltpu.async_copy(p_hbm.at[idxrf_v.at[pl.ds(cc * _CH, _CH)]], bp, semp)
    pltpu.async_copy(q_hbm.at[idxcf_v.at[pl.ds(cc * _CH, _CH)]], bq, semq)

  def waitg(bp, bq, semp, semq):
    pltpu.make_async_copy(p_hbm.at[idxrf_v.at[pl.ds(0, _CH)]], bp,
                          semp).wait()
    pltpu.make_async_copy(q_hbm.at[idxcf_v.at[pl.ds(0, _CH)]], bq,
                          semq).wait()

  def coordwork(cc):
    lane = lax.iota(jnp.int32, 16)
    col0 = jnp.zeros((16,), jnp.int32)
    for k in range(_CH // 16):
      m = cc * _CH + k * 16
      ir = idxrf_v[pl.ds(m, 16)]
      ic = idxcf_v[pl.ds(m, 16)]
      dx = plsc.load_gather(cx_v, [ir]) - plsc.load_gather(cx_v, [ic])
      dy = plsc.load_gather(cy_v, [ir]) - plsc.load_gather(cy_v, [ic])
      dz = plsc.load_gather(cz_v, [ir]) - plsc.load_gather(cz_v, [ic])
      rad = dx * dx + dy * dy + dz * dz
      rows = lane + (k * 16)
      plsc.store_scatter(cdbuf, [rows, col0], dx)
      plsc.store_scatter(cdbuf, [rows, col0 + 1], dy)
      plsc.store_scatter(cdbuf, [rows, col0 + 2], dz)
      plsc.store_scatter(cdbuf, [rows, col0 + 3], rad)

  def addbufs(bp, bq):
    def arow(i, carry):
      for k in range(_D // 16):
        sl = pl.ds(k * 16, 16)
        bp[i, sl] = bp[i, sl] + bq[i, sl]
      return carry

    lax.fori_loop(0, _CH, arow, 0)

  def emit(cc, bp, bq):
    addbufs(bp, bq)
    row0 = base + cc * _CH
    pltpu.sync_copy(bp, g_hbm.at[pl.ds(row0, _CH), :])
    pltpu.sync_copy(cdbuf, cd_hbm.at[pl.ds(row0, _CH), :])

  issue(0, b0p, b0q, sem0, sem1)

  def body(j2, carry):
    c0 = 2 * j2
    issue(c0 + 1, b1p, b1q, sem2, sem3)
    coordwork(c0)
    waitg(b0p, b0q, sem0, sem1)
    emit(c0, b0p, b0q)
    issue(c0 + 2, b0p, b0q, sem0, sem1)
    coordwork(c0 + 1)
    waitg(b1p, b1q, sem2, sem3)
    emit(c0 + 1, b1p, b1q)
    return carry

  lax.fori_loop(0, (_NCH - 1) // 2, body, 0)
  coordwork(_NCH - 1)
  waitg(b0p, b0q, sem0, sem1)
  emit(_NCH - 1, b0p, b0q)


def _sc_gather(p, q, cx, cy, cz, idxrf, idxcf):
  mesh = plsc.VectorSubcoreMesh(core_axis_name="c", subcore_axis_name="s")
  f = pl.kernel(
      _sc_gather_body,
      out_type=[
          jax.ShapeDtypeStruct((_E, _D), jnp.float32),
          jax.ShapeDtypeStruct((_E, _CP), jnp.float32),
      ],
      mesh=mesh,
      scratch_types=[
          pltpu.VMEM((_TPW,), jnp.int32),
          pltpu.VMEM((_TPW,), jnp.int32),
          pltpu.VMEM((_N,), jnp.float32),
          pltpu.VMEM((_N,), jnp.float32),
          pltpu.VMEM((_N,), jnp.float32),
          pltpu.VMEM((2, _CH, _D), jnp.float32),
          pltpu.VMEM((2, _CH, _D), jnp.float32),
          pltpu.VMEM((_CH, _CP), jnp.float32),
          pltpu.SemaphoreType.DMA,
          pltpu.SemaphoreType.DMA,
          pltpu.SemaphoreType.DMA,
          pltpu.SemaphoreType.DMA,
      ],
      compiler_params=pltpu.CompilerParams(needs_layout_passes=False),
  )
  return f(p, q, cx, cy, cz, idxrf, idxcf)


def _make_scatter_body(nrows, nslab):
  w = min(_SCH, nslab)

  def body_fn(src_hbm, idx_hbm, z_hbm, out_hbm, idx_v, buf, acc_sh,
              sem0, sem1):
    c = lax.axis_index("c")
    s = lax.axis_index("s")
    wid = s * _NC + c
    base = wid * _TPW
    slab = s * nslab
    pltpu.sync_copy(z_hbm.at[pl.ds(0, _SCH), :], buf.at[0])

    def zbody(k, carry):
      off = slab + k * _SCH
      pltpu.sync_copy(buf.at[0, pl.ds(0, w), :],
                      acc_sh.at[pl.ds(off, w), :])
      return carry

    lax.fori_loop(0, max(1, nslab // _SCH), zbody, 0)
    pltpu.sync_copy(idx_hbm.at[wid], idx_v)
    plsc.subcore_barrier()

    def sissue(cc, bslot, sem):
      row0 = base + cc * _SCH
      pltpu.async_copy(src_hbm.at[pl.ds(row0, _SCH), :], bslot, sem)

    def swait(bslot, sem):
      pltpu.make_async_copy(src_hbm.at[pl.ds(0, _SCH), :], bslot, sem).wait()

    sissue(0, buf.at[0], sem0)

    def body(j2, carry):
      c0 = 2 * j2
      sissue(c0 + 1, buf.at[1], sem1)
      swait(buf.at[0], sem0)
      pltpu.sync_copy(buf.at[0], acc_sh.at[idx_v.at[c0]], add=True)
      sissue(c0 + 2, buf.at[0], sem0)
      swait(buf.at[1], sem1)
      pltpu.sync_copy(buf.at[1], acc_sh.at[idx_v.at[c0 + 1]], add=True)
      return carry

    lax.fori_loop(0, (_SNCH - 1) // 2, body, 0)
    if _SNCH % 2 == 0:
      sissue(_SNCH - 1, buf.at[1], sem1)
      swait(buf.at[0], sem0)
      pltpu.sync_copy(buf.at[0], acc_sh.at[idx_v.at[_SNCH - 2]], add=True)
      swait(buf.at[1], sem1)
      pltpu.sync_copy(buf.at[1], acc_sh.at[idx_v.at[_SNCH - 1]], add=True)
    else:
      swait(buf.at[0], sem0)
      pltpu.sync_copy(buf.at[0], acc_sh.at[idx_v.at[_SNCH - 1]], add=True)
    plsc.subcore_barrier()

    def wbody(k, carry):
      off = slab + k * _SCH
      pltpu.sync_copy(acc_sh.at[pl.ds(off, w), :], buf.at[0, pl.ds(0, w), :])
      pltpu.sync_copy(buf.at[0, pl.ds(0, w), :],
                      out_hbm.at[c, pl.ds(off, w), :])
      return carry

    lax.fori_loop(0, max(1, nslab // _SCH), wbody, 0)

  return body_fn


def _sc_scatter(src, idx3, z128, nrows, nslab):
  mesh = plsc.VectorSubcoreMesh(core_axis_name="c", subcore_axis_name="s")
  f = pl.kernel(
      _make_scatter_body(nrows, nslab),
      out_type=[
          jax.ShapeDtypeStruct((_NC, nrows, _H), jnp.float32),
      ],
      mesh=mesh,
      scratch_types=[
          pltpu.VMEM((_SNCH, _SCH), jnp.int32),
          pltpu.VMEM((2, _SCH, _H), jnp.float32),
          pltpu.VMEM_SHARED((nrows, _H), jnp.float32),
          pltpu.SemaphoreType.DMA,
          pltpu.SemaphoreType.DMA,
      ],
  )
  return f(src, idx3, z128)[0]


_BN = 1000
_BE = 1280


def _prep_body(h_ref, wa_ref, wb_ref, wr_ref, p_ref, q_ref, r_ref):
  h = h_ref[...]
  p_ref[...] = jnp.dot(h, wa_ref[...], preferred_element_type=jnp.float32)
  q_ref[...] = jnp.dot(h, wb_ref[...], preferred_element_type=jnp.float32)
  r_ref[...] = jnp.dot(h, wr_ref[...], preferred_element_type=jnp.float32)


def _tc_prep(h, wa, wb, wr):
  grid = (_N // _BN,)
  return pl.pallas_call(
      _prep_body,
      grid=grid,
      in_specs=[
          pl.BlockSpec((_BN, _D), lambda i: (i, 0)),
          pl.BlockSpec((_D, _H), lambda i: (0, 0)),
          pl.BlockSpec((_D, _H), lambda i: (0, 0)),
          pl.BlockSpec((_D, _H), lambda i: (0, 0)),
      ],
      out_specs=[
          pl.BlockSpec((_BN, _H), lambda i: (i, 0)),
          pl.BlockSpec((_BN, _H), lambda i: (i, 0)),
          pl.BlockSpec((_BN, _H), lambda i: (i, 0)),
      ],
      out_shape=[
          jax.ShapeDtypeStruct((_N, _H), jnp.float32),
          jax.ShapeDtypeStruct((_N, _H), jnp.float32),
          jax.ShapeDtypeStruct((_N, _H), jnp.float32),
      ],
  )(h, wa, wb, wr)


def _silu(x):
  return x * jax.nn.sigmoid(x)


def _edge_body(g_ref, cd_ref, ea_ref, rem_ref,
               w1d_ref, w1c_ref, be1_ref, we2_ref, be2_ref,
               wc1_ref, bc1_ref, wc2t_ref,
               ef_ref, tp_ref):
  cd = cd_ref[...]
  lanes = lax.broadcasted_iota(jnp.int32, (1, _CP), 1)
  diff = jnp.where(lanes < 3, cd, 0.0)
  radial = jnp.sum(jnp.where(lanes == 3, cd, 0.0),
                   axis=1, keepdims=True)
  norm = jnp.sqrt(radial + 1e-8)
  pre1 = (g_ref[...]
          + radial * w1c_ref[...]
          + jnp.dot(ea_ref[...], w1d_ref[...],
                    preferred_element_type=jnp.float32)
          + be1_ref[...])
  f = _silu(pre1)
  ef = _silu(jnp.dot(f, we2_ref[...], preferred_element_type=jnp.float32)
             + be2_ref[...])
  m = _silu(jnp.dot(ef, wc1_ref[...], preferred_element_type=jnp.float32)
            + bc1_ref[...])
  csc = jnp.sum(m * wc2t_ref[...], axis=1, keepdims=True)
  ef_ref[...] = ef
  trans = diff * (csc / norm)
  big = jnp.concatenate([trans[:, :_CW]] * 16, axis=1)
  grp = lax.broadcasted_iota(jnp.int32, (1, _D), 1) // _CW
  tp_ref[...] = jnp.where(grp == rem_ref[...], big, 0.0)


def _tc_edge(g, cd, ea, rem, w1d, w1c, be1, we2, be2, wc1, bc1, wc2t):
  grid = (_E // _BE,)
  return pl.pallas_call(
      _edge_body,
      grid=grid,
      in_specs=[
          pl.BlockSpec((_BE, _D), lambda i: (i, 0)),
          pl.BlockSpec((_BE, _CP), lambda i: (i, 0)),
          pl.BlockSpec((_BE, _DE), lambda i: (i, 0)),
          pl.BlockSpec((_BE, 1), lambda i: (i, 0)),
          pl.BlockSpec((_DE, _H), lambda i: (0, 0)),
          pl.BlockSpec((1, _H), lambda i: (0, 0)),
          pl.BlockSpec((1, _H), lambda i: (0, 0)),
          pl.BlockSpec((_H, _H), lambda i: (0, 0)),
          pl.BlockSpec((1, _H), lambda i: (0, 0)),
          pl.BlockSpec((_H, _H), lambda i: (0, 0)),
          pl.BlockSpec((1, _H), lambda i: (0, 0)),
          pl.BlockSpec((1, _H), lambda i: (0, 0)),
      ],
      out_specs=[
          pl.BlockSpec((_BE, _H), lambda i: (i, 0)),
          pl.BlockSpec((_BE, _D), lambda i: (i, 0)),
      ],
      out_shape=[
          jax.ShapeDtypeStruct((_E, _H), jnp.float32),
          jax.ShapeDtypeStruct((_E, _D), jnp.float32),
      ],
  )(g, cd, ea, rem, w1d, w1c, be1, we2, be2, wc1, bc1, wc2t)


def _node_body(h_ref, r_ref, agg0_ref, agg1_ref, ca0_ref, ca1_ref, cp_ref,
               wn1b_ref, bn1_ref, wn2_ref, bn2_ref,
               hout_ref, cpout_ref):
  agg = agg0_ref[...] + agg1_ref[...]
  mid = _silu(r_ref[...]
              + jnp.dot(agg, wn1b_ref[...], preferred_element_type=jnp.float32)
              + bn1_ref[...])
  hout_ref[...] = (h_ref[...]
                   + jnp.dot(mid, wn2_ref[...],
                             preferred_element_type=jnp.float32)
                   + bn2_ref[...])
  cpout_ref[...] = cp_ref[...] + ca0_ref[...] + ca1_ref[...]


def _tc_node(h, r, agg0, agg1, ca0, ca1, cp, wn1b, bn1, wn2, bn2):
  grid = (_N // _BN,)
  return pl.pallas_call(
      _node_body,
      grid=grid,
      in_specs=[
          pl.BlockSpec((_BN, _D), lambda i: (i, 0)),
          pl.BlockSpec((_BN, _H), lambda i: (i, 0)),
          pl.BlockSpec((_BN, _H), lambda i: (i, 0)),
          pl.BlockSpec((_BN, _H), lambda i: (i, 0)),
          pl.BlockSpec((_BN, _CW), lambda i: (i, 0)),
          pl.BlockSpec((_BN, _CW), lambda i: (i, 0)),
          pl.BlockSpec((_BN, _CW), lambda i: (i, 0)),
          pl.BlockSpec((_H, _H), lambda i: (0, 0)),
          pl.BlockSpec((1, _H), lambda i: (0, 0)),
          pl.BlockSpec((_H, _D), lambda i: (0, 0)),
          pl.BlockSpec((1, _D), lambda i: (0, 0)),
      ],
      out_specs=[
          pl.BlockSpec((_BN, _D), lambda i: (i, 0)),
          pl.BlockSpec((_BN, _CW), lambda i: (i, 0)),
      ],
      out_shape=[
          jax.ShapeDtypeStruct((_N, _D), jnp.float32),
          jax.ShapeDtypeStruct((_N, _CW), jnp.float32),
      ],
  )(h, r, agg0, agg1, ca0, ca1, cp, wn1b, bn1, wn2, bn2)


@jax.jit
def kernel(h, edge_index, coord, edge_attr,
           W_e1, b_e1, W_e2, b_e2,
           W_n1, b_n1, W_n2, b_n2,
           W_c1, b_c1, W_c2):
  row = edge_index[0]
  col = edge_index[1]
  idxrf = row.reshape(_NW, _TPW)
  idxcf = col.reshape(_NW, _TPW)
  coordp = jnp.zeros((_N, _CW), jnp.float32).at[:, :3].set(coord)
  cx = coord[:, 0]
  cy = coord[:, 1]
  cz = coord[:, 2]

  p, q, r = _tc_prep(h, W_e1[:_D], W_e1[_D:2 * _D], W_n1[:_D])
  g, cd = _sc_gather(p, q, cx, cy, cz, idxrf, idxcf)

  w1c = W_e1[2 * _D:2 * _D + 1]
  w1d = W_e1[2 * _D + 1:]
  rem = (row % 16).reshape(_E, 1)
  ef, tp = _tc_edge(g, cd, edge_attr, rem,
                    w1d, w1c, b_e1[None, :], W_e2, b_e2[None, :],
                    W_c1, b_c1[None, :], W_c2.reshape(1, _H))

  idxr_s = row.reshape(_NW, _SNCH, _SCH)
  idxq_s = (row // 16).reshape(_NW, _SNCH, _SCH)
  z128 = jnp.zeros((_NP, _H), jnp.float32)
  aggp = _sc_scatter(ef, idxr_s, z128, _NP, _NPS)
  caggp = _sc_scatter(tp, idxq_s, z128, _NQ, _NQS)
  cagg = caggp.reshape(_NC, _NP, _CW)

  h_out, cpout = _tc_node(h, r, aggp[0], aggp[1], cagg[0], cagg[1], coordp,
                          W_n1[_D:], b_n1[None, :], W_n2, b_n2[None, :])
  coord_out = cpout[:, :3]
  return (h_out, coord_out, edge_attr)


# speedup vs baseline: 1.0716x; 1.0716x over previous
import functools

import jax
import jax.numpy as jnp
from jax import lax
from jax.experimental import pallas as pl
from jax.experimental.pallas import tpu as pltpu
from jax.experimental.pallas import tpu_sc as plsc

_N = 10000
_E = 320000
_D = 128
_H = 128
_DE = 16
_CP = 16

_NC = 2
_NS = 16
_NW = _NC * _NS
_TPW = _E // _NW
_CH = 80
_NCH = _TPW // _CH
_NP = 10240
_NPS = _NP // _NS
_SCH = 40
_SNCH = _TPW // _SCH
_CW = 8
_NQ = _NP // 16
_NQS = _NQ // _NS


def _sc_gather_body(p_hbm, q_hbm, cx_hbm, cy_hbm, cz_hbm,
                    idxrf_hbm, idxcf_hbm,
                    g_hbm, cd_hbm,
                    idxrf_v, idxcf_v,
                    cx_v, cy_v, cz_v, bufp, bufq, cdbuf,
                    sem0, sem1, sem2, sem3):
  c = lax.axis_index("c")
  s = lax.axis_index("s")
  wid = s * _NC + c
  base = wid * _TPW
  pltpu.sync_copy(idxrf_hbm.at[wid], idxrf_v)
  pltpu.sync_copy(idxcf_hbm.at[wid], idxcf_v)
  pltpu.sync_copy(cx_hbm, cx_v)
  pltpu.sync_copy(cy_hbm, cy_v)
  pltpu.sync_copy(cz_hbm, cz_v)

  b0p, b0q = bufp.at[0], bufq.at[0]
  b1p, b1q = bufp.at[1], bufq.at[1]

  def issue(cc, bp, bq, semp, semq):
    pltpu.async_copy(p_hbm.at[idxrf_v.at[pl.ds(cc * _CH, _CH)]], bp, semp)
    pltpu.async_copy(q_hbm.at[idxcf_v.at[pl.ds(cc * _CH, _CH)]], bq, semq)

  def waitg(bp, bq, semp, semq):
    pltpu.make_async_copy(p_hbm.at[idxrf_v.at[pl.ds(0, _CH)]], bp,
                          semp).wait()
    pltpu.make_async_copy(q_hbm.at[idxcf_v.at[pl.ds(0, _CH)]], bq,
                          semq).wait()

  def coordwork(cc):
    lane = lax.iota(jnp.int32, 16)
    col0 = jnp.zeros((16,), jnp.int32)
    for k in range(_CH // 16):
      m = cc * _CH + k * 16
      ir = idxrf_v[pl.ds(m, 16)]
      ic = idxcf_v[pl.ds(m, 16)]
      dx = plsc.load_gather(cx_v, [ir]) - plsc.load_gather(cx_v, [ic])
      dy = plsc.load_gather(cy_v, [ir]) - plsc.load_gather(cy_v, [ic])
      dz = plsc.load_gather(cz_v, [ir]) - plsc.load_gather(cz_v, [ic])
      rad = dx * dx + dy * dy + dz * dz
      rows = lane + (k * 16)
      plsc.store_scatter(cdbuf, [rows, col0], dx)
      plsc.store_scatter(cdbuf, [rows, col0 + 1], dy)
      plsc.store_scatter(cdbuf, [rows, col0 + 2], dz)
      plsc.store_scatter(cdbuf, [rows, col0 + 3], rad)

  def addbufs(bp, bq):
    def arow(i, carry):
      for k in range(_D // 16):
        sl = pl.ds(k * 16, 16)
        bp[i, sl] = bp[i, sl] + bq[i, sl]
      return carry

    lax.fori_loop(0, _CH, arow, 0)

  def emit(cc, bp, bq):
    addbufs(bp, bq)
    row0 = base + cc * _CH
    pltpu.sync_copy(bp, g_hbm.at[pl.ds(row0, _CH), :])
    pltpu.sync_copy(cdbuf, cd_hbm.at[pl.ds(row0, _CH), :])

  issue(0, b0p, b0q, sem0, sem1)

  def body(j2, carry):
    c0 = 2 * j2
    issue(c0 + 1, b1p, b1q, sem2, sem3)
    coordwork(c0)
    waitg(b0p, b0q, sem0, sem1)
    emit(c0, b0p, b0q)
    issue(c0 + 2, b0p, b0q, sem0, sem1)
    coordwork(c0 + 1)
    waitg(b1p, b1q, sem2, sem3)
    emit(c0 + 1, b1p, b1q)
    return carry

  lax.fori_loop(0, (_NCH - 1) // 2, body, 0)
  coordwork(_NCH - 1)
  waitg(b0p, b0q, sem0, sem1)
  emit(_NCH - 1, b0p, b0q)


def _sc_gather(p, q, cx, cy, cz, idxrf, idxcf):
  mesh = plsc.VectorSubcoreMesh(core_axis_name="c", subcore_axis_name="s")
  f = pl.kernel(
      _sc_gather_body,
      out_type=[
          jax.ShapeDtypeStruct((_E, _D), jnp.float32),
          jax.ShapeDtypeStruct((_E, _CP), jnp.float32),
      ],
      mesh=mesh,
      scratch_types=[
          pltpu.VMEM((_TPW,), jnp.int32),
          pltpu.VMEM((_TPW,), jnp.int32),
          pltpu.VMEM((_N,), jnp.float32),
          pltpu.VMEM((_N,), jnp.float32),
          pltpu.VMEM((_N,), jnp.float32),
          pltpu.VMEM((2, _CH, _D), jnp.float32),
          pltpu.VMEM((2, _CH, _D), jnp.float32),
          pltpu.VMEM((_CH, _CP), jnp.float32),
          pltpu.SemaphoreType.DMA,
          pltpu.SemaphoreType.DMA,
          pltpu.SemaphoreType.DMA,
          pltpu.SemaphoreType.DMA,
      ],
      compiler_params=pltpu.CompilerParams(needs_layout_passes=False),
  )
  return f(p, q, cx, cy, cz, idxrf, idxcf)


def _make_scatter_body(nrows, nslab):
  w = min(_SCH, nslab)

  def body_fn(src_hbm, idx_hbm, z_hbm, out_hbm, idx_v, buf, acc_sh,
              sem0, sem1):
    c = lax.axis_index("c")
    s = lax.axis_index("s")
    wid = s * _NC + c
    base = wid * _TPW
    slab = s * nslab
    pltpu.sync_copy(z_hbm.at[pl.ds(0, _SCH), :], buf.at[0])

    def zbody(k, carry):
      off = slab + k * _SCH
      pltpu.sync_copy(buf.at[0, pl.ds(0, w), :],
                      acc_sh.at[pl.ds(off, w), :])
      return carry

    lax.fori_loop(0, max(1, nslab // _SCH), zbody, 0)
    pltpu.sync_copy(idx_hbm.at[wid], idx_v)
    plsc.subcore_barrier()

    def sissue(cc, bslot, sem):
      row0 = base + cc * _SCH
      pltpu.async_copy(src_hbm.at[pl.ds(row0, _SCH), :], bslot, sem)

    def swait(bslot, sem):
      pltpu.make_async_copy(src_hbm.at[pl.ds(0, _SCH), :], bslot, sem).wait()

    sissue(0, buf.at[0], sem0)

    def body(j2, carry):
      c0 = 2 * j2
      sissue(c0 + 1, buf.at[1], sem1)
      swait(buf.at[0], sem0)
      pltpu.sync_copy(buf.at[0], acc_sh.at[idx_v.at[c0]], add=True)
      sissue(c0 + 2, buf.at[0], sem0)
      swait(buf.at[1], sem1)
      pltpu.sync_copy(buf.at[1], acc_sh.at[idx_v.at[c0 + 1]], add=True)
      return carry

    lax.fori_loop(0, (_SNCH - 1) // 2, body, 0)
    if _SNCH % 2 == 0:
      sissue(_SNCH - 1, buf.at[1], sem1)
      swait(buf.at[0], sem0)
      pltpu.sync_copy(buf.at[0], acc_sh.at[idx_v.at[_SNCH - 2]], add=True)
      swait(buf.at[1], sem1)
      pltpu.sync_copy(buf.at[1], acc_sh.at[idx_v.at[_SNCH - 1]], add=True)
    else:
      swait(buf.at[0], sem0)
      pltpu.sync_copy(buf.at[0], acc_sh.at[idx_v.at[_SNCH - 1]], add=True)
    plsc.subcore_barrier()

    def wbody(k, carry):
      off = slab + k * _SCH
      pltpu.sync_copy(acc_sh.at[pl.ds(off, w), :], buf.at[0, pl.ds(0, w), :])
      pltpu.sync_copy(buf.at[0, pl.ds(0, w), :],
                      out_hbm.at[c, pl.ds(off, w), :])
      return carry

    lax.fori_loop(0, max(1, nslab // _SCH), wbody, 0)

  return body_fn


def _sc_scatter(src, idx3, z128, nrows, nslab):
  mesh = plsc.VectorSubcoreMesh(core_axis_name="c", subcore_axis_name="s")
  f = pl.kernel(
      _make_scatter_body(nrows, nslab),
      out_type=[
          jax.ShapeDtypeStruct((_NC, nrows, _H), jnp.float32),
      ],
      mesh=mesh,
      scratch_types=[
          pltpu.VMEM((_SNCH, _SCH), jnp.int32),
          pltpu.VMEM((2, _SCH, _H), jnp.float32),
          pltpu.VMEM_SHARED((nrows, _H), jnp.float32),
          pltpu.SemaphoreType.DMA,
          pltpu.SemaphoreType.DMA,
      ],
  )
  return f(src, idx3, z128)[0]


_BN = 1000
_BE = 640


def _prep_body(h_ref, wa_ref, wb_ref, wr_ref, p_ref, q_ref, r_ref):
  h = h_ref[...]
  p_ref[...] = jnp.dot(h, wa_ref[...], preferred_element_type=jnp.float32)
  q_ref[...] = jnp.dot(h, wb_ref[...], preferred_element_type=jnp.float32)
  r_ref[...] = jnp.dot(h, wr_ref[...], preferred_element_type=jnp.float32)


def _tc_prep(h, wa, wb, wr):
  grid = (_N // _BN,)
  return pl.pallas_call(
      _prep_body,
      grid=grid,
      in_specs=[
          pl.BlockSpec((_BN, _D), lambda i: (i, 0)),
          pl.BlockSpec((_D, _H), lambda i: (0, 0)),
          pl.BlockSpec((_D, _H), lambda i: (0, 0)),
          pl.BlockSpec((_D, _H), lambda i: (0, 0)),
      ],
      out_specs=[
          pl.BlockSpec((_BN, _H), lambda i: (i, 0)),
          pl.BlockSpec((_BN, _H), lambda i: (i, 0)),
          pl.BlockSpec((_BN, _H), lambda i: (i, 0)),
      ],
      out_shape=[
          jax.ShapeDtypeStruct((_N, _H), jnp.float32),
          jax.ShapeDtypeStruct((_N, _H), jnp.float32),
          jax.ShapeDtypeStruct((_N, _H), jnp.float32),
      ],
  )(h, wa, wb, wr)


def _silu(x):
  return x * jax.nn.sigmoid(x)


def _edge_body(g_ref, cd_ref, ea_ref, rem_ref,
               w1d_ref, w1c_ref, be1_ref, we2_ref, be2_ref,
               wc1_ref, bc1_ref, wc2t_ref,
               ef_ref, tp_ref):
  cd = cd_ref[...]
  lanes = lax.broadcasted_iota(jnp.int32, (1, _CP), 1)
  diff = jnp.where(lanes < 3, cd, 0.0)
  radial = jnp.sum(jnp.where(lanes == 3, cd, 0.0),
                   axis=1, keepdims=True)
  norm = jnp.sqrt(radial + 1e-8)
  pre1 = (g_ref[...]
          + radial * w1c_ref[...]
          + jnp.dot(ea_ref[...], w1d_ref[...],
                    preferred_element_type=jnp.float32)
          + be1_ref[...])
  f = _silu(pre1)
  ef = _silu(jnp.dot(f, we2_ref[...], preferred_element_type=jnp.float32)
             + be2_ref[...])
  m = _silu(jnp.dot(ef, wc1_ref[...], preferred_element_type=jnp.float32)
            + bc1_ref[...])
  csc = jnp.sum(m * wc2t_ref[...], axis=1, keepdims=True)
  ef_ref[...] = ef
  trans = diff * (csc / norm)
  big = jnp.concatenate([trans[:, :_CW]] * 16, axis=1)
  grp = lax.broadcasted_iota(jnp.int32, (1, _D), 1) // _CW
  tp_ref[...] = jnp.where(grp == rem_ref[...], big, 0.0)


def _tc_edge(g, cd, ea, rem, w1d, w1c, be1, we2, be2, wc1, bc1, wc2t):
  grid = (_E // _BE,)
  return pl.pallas_call(
      _edge_body,
      grid=grid,
      in_specs=[
          pl.BlockSpec((_BE, _D), lambda i: (i, 0)),
          pl.BlockSpec((_BE, _CP), lambda i: (i, 0)),
          pl.BlockSpec((_BE, _DE), lambda i: (i, 0)),
          pl.BlockSpec((_BE, 1), lambda i: (i, 0)),
          pl.BlockSpec((_DE, _H), lambda i: (0, 0)),
          pl.BlockSpec((1, _H), lambda i: (0, 0)),
          pl.BlockSpec((1, _H), lambda i: (0, 0)),
          pl.BlockSpec((_H, _H), lambda i: (0, 0)),
          pl.BlockSpec((1, _H), lambda i: (0, 0)),
          pl.BlockSpec((_H, _H), lambda i: (0, 0)),
          pl.BlockSpec((1, _H), lambda i: (0, 0)),
          pl.BlockSpec((1, _H), lambda i: (0, 0)),
      ],
      out_specs=[
          pl.BlockSpec((_BE, _H), lambda i: (i, 0)),
          pl.BlockSpec((_BE, _D), lambda i: (i, 0)),
      ],
      out_shape=[
          jax.ShapeDtypeStruct((_E, _H), jnp.float32),
          jax.ShapeDtypeStruct((_E, _D), jnp.float32),
      ],
  )(g, cd, ea, rem, w1d, w1c, be1, we2, be2, wc1, bc1, wc2t)


def _node_body(h_ref, r_ref, agg0_ref, agg1_ref, ca0_ref, ca1_ref, cp_ref,
               wn1b_ref, bn1_ref, wn2_ref, bn2_ref,
               hout_ref, cpout_ref):
  agg = agg0_ref[...] + agg1_ref[...]
  mid = _silu(r_ref[...]
              + jnp.dot(agg, wn1b_ref[...], preferred_element_type=jnp.float32)
              + bn1_ref[...])
  hout_ref[...] = (h_ref[...]
                   + jnp.dot(mid, wn2_ref[...],
                             preferred_element_type=jnp.float32)
                   + bn2_ref[...])
  cpout_ref[...] = cp_ref[...] + ca0_ref[...] + ca1_ref[...]


def _tc_node(h, r, agg0, agg1, ca0, ca1, cp, wn1b, bn1, wn2, bn2):
  grid = (_N // _BN,)
  return pl.pallas_call(
      _node_body,
      grid=grid,
      in_specs=[
          pl.BlockSpec((_BN, _D), lambda i: (i, 0)),
          pl.BlockSpec((_BN, _H), lambda i: (i, 0)),
          pl.BlockSpec((_BN, _H), lambda i: (i, 0)),
          pl.BlockSpec((_BN, _H), lambda i: (i, 0)),
          pl.BlockSpec((_BN, _CW), lambda i: (i, 0)),
          pl.BlockSpec((_BN, _CW), lambda i: (i, 0)),
          pl.BlockSpec((_BN, _CW), lambda i: (i, 0)),
          pl.BlockSpec((_H, _H), lambda i: (0, 0)),
          pl.BlockSpec((1, _H), lambda i: (0, 0)),
          pl.BlockSpec((_H, _D), lambda i: (0, 0)),
          pl.BlockSpec((1, _D), lambda i: (0, 0)),
      ],
      out_specs=[
          pl.BlockSpec((_BN, _D), lambda i: (i, 0)),
          pl.BlockSpec((_BN, _CW), lambda i: (i, 0)),
      ],
      out_shape=[
          jax.ShapeDtypeStruct((_N, _D), jnp.float32),
          jax.ShapeDtypeStruct((_N, _CW), jnp.float32),
      ],
  )(h, r, agg0, agg1, ca0, ca1, cp, wn1b, bn1, wn2, bn2)


@jax.jit
def kernel(h, edge_index, coord, edge_attr,
           W_e1, b_e1, W_e2, b_e2,
           W_n1, b_n1, W_n2, b_n2,
           W_c1, b_c1, W_c2):
  row = edge_index[0]
  col = edge_index[1]
  idxrf = row.reshape(_NW, _TPW)
  idxcf = col.reshape(_NW, _TPW)
  coordp = jnp.zeros((_N, _CW), jnp.float32).at[:, :3].set(coord)
  cx = coord[:, 0]
  cy = coord[:, 1]
  cz = coord[:, 2]

  p, q, r = _tc_prep(h, W_e1[:_D], W_e1[_D:2 * _D], W_n1[:_D])
  g, cd = _sc_gather(p, q, cx, cy, cz, idxrf, idxcf)

  w1c = W_e1[2 * _D:2 * _D + 1]
  w1d = W_e1[2 * _D + 1:]
  rem = (row % 16).reshape(_E, 1)
  ef, tp = _tc_edge(g, cd, edge_attr, rem,
                    w1d, w1c, b_e1[None, :], W_e2, b_e2[None, :],
                    W_c1, b_c1[None, :], W_c2.reshape(1, _H))

  idxr_s = row.reshape(_NW, _SNCH, _SCH)
  idxq_s = (row // 16).reshape(_NW, _SNCH, _SCH)
  z128 = jnp.zeros((_NP, _H), jnp.float32)
  aggp = _sc_scatter(ef, idxr_s, z128, _NP, _NPS)
  caggp = _sc_scatter(tp, idxq_s, z128, _NQ, _NQS)
  cagg = caggp.reshape(_NC, _NP, _CW)

  h_out, cpout = _tc_node(h, r, aggp[0], aggp[1], cagg[0], cagg[1], coordp,
                          W_n1[_D:], b_n1[None, :], W_n2, b_n2[None, :])
  coord_out = cpout[:, :3]
  return (h_out, coord_out, edge_attr)
